# Initial kernel scaffold; baseline (speedup 1.0000x reference)
import jax
import jax.numpy as jnp
from jax.experimental import pallas as pl


def kernel(x, edge_index, fake_x, fake_edge_index, treat_idx, control_idx, W1, a_src1, a_dst1, b1, W2, a_src2, a_dst2, b2, Wy1, by1, Wy0, by0, Wb, bb, Wp1, bp1, Wp2, bp2):
    raise NotImplementedError("write your pallas kernel here")


# speedup vs baseline: 12.3207x; 12.3207x over previous
import functools

import jax
import jax.numpy as jnp
from jax import lax
from jax.experimental import pallas as pl
from jax.experimental.pallas import tpu as pltpu
from jax.experimental.pallas import tpu_sc as plsc

N = 10000
M = 2 * N
EC = 320000
D = 64
DP = 128
IN_DIM = 128

NC, NS = 2, 16
NW = NC * NS
CH = 128
NCHUNK = 160
EPW = NCHUNK * CH
EPC = NS * EPW
NTOT = NC * EPC
N_PAD = 10240
ROWS_PS = N_PAD // NS
N_IDX = 5000
G_PAD = 5120
G4 = 4 * G_PAD

_f32 = jnp.float32


RB = 2000
RBE = 4096


def _lrelu(x, slope):
    return jnp.where(x > 0, x, slope * x)


def _pack_tab(h, s, d, rb):
    col = lax.broadcasted_iota(jnp.int32, (rb, DP), 1)
    return jnp.where(col == D, s, jnp.where(col == D + 1, d, h))


def _dense1_body(x_ref, w_ref, as_ref, ad_ref, t_ref, ws_ref):
    h = jnp.dot(x_ref[...], w_ref[...], preferred_element_type=_f32)
    s = jnp.sum(h * as_ref[...], axis=1, keepdims=True)
    d = jnp.sum(h * ad_ref[...], axis=1, keepdims=True)
    t_ref[...] = _pack_tab(h, s, d, RB)
    ws_ref[...] = jnp.exp(_lrelu(s + d, 0.2))


_dense1 = pl.pallas_call(
    _dense1_body,
    grid=(M // RB,),
    in_specs=[
        pl.BlockSpec((RB, IN_DIM), lambda i: (i, 0)),
        pl.BlockSpec((IN_DIM, DP), lambda i: (0, 0)),
        pl.BlockSpec((1, DP), lambda i: (0, 0)),
        pl.BlockSpec((1, DP), lambda i: (0, 0)),
    ],
    out_specs=[
        pl.BlockSpec((RB, DP), lambda i: (i, 0)),
        pl.BlockSpec((RB, 1), lambda i: (i, 0)),
    ],
    out_shape=[
        jax.ShapeDtypeStruct((M, DP), _f32),
        jax.ShapeDtypeStruct((M, 1), _f32),
    ],
)


def _dense2_body(agg_ref, dp_ref, h1_ref, ws_ref, b_ref, w_ref,
                 as_ref, ad_ref, t_ref, ws2_ref):
    rd = 1.0 / (dp_ref[...] + ws_ref[...] + 1e-16)
    z = rd * (agg_ref[...] + ws_ref[...] * h1_ref[...][:, :D]) + b_ref[...]
    z = jnp.maximum(z, 0.0)
    h = jnp.dot(z, w_ref[...], preferred_element_type=_f32)
    s = jnp.sum(h * as_ref[...], axis=1, keepdims=True)
    d = jnp.sum(h * ad_ref[...], axis=1, keepdims=True)
    t_ref[...] = _pack_tab(h, s, d, RB)
    ws2_ref[...] = jnp.exp(_lrelu(s + d, 0.2))


_dense2 = pl.pallas_call(
    _dense2_body,
    grid=(M // RB,),
    in_specs=[
        pl.BlockSpec((RB, D), lambda i: (i, 0)),
        pl.BlockSpec((RB, 1), lambda i: (i, 0)),
        pl.BlockSpec((RB, DP), lambda i: (i, 0)),
        pl.BlockSpec((RB, 1), lambda i: (i, 0)),
        pl.BlockSpec((1, D), lambda i: (0, 0)),
        pl.BlockSpec((D, DP), lambda i: (0, 0)),
        pl.BlockSpec((1, DP), lambda i: (0, 0)),
        pl.BlockSpec((1, DP), lambda i: (0, 0)),
    ],
    out_specs=[
        pl.BlockSpec((RB, DP), lambda i: (i, 0)),
        pl.BlockSpec((RB, 1), lambda i: (i, 0)),
    ],
    out_shape=[
        jax.ShapeDtypeStruct((M, DP), _f32),
        jax.ShapeDtypeStruct((M, 1), _f32),
    ],
)


def _escale_body(hs_ref, hd_ref, o_ref):
    hs = hs_ref[...]
    hd = hd_ref[...]
    a = hs[:, D:D + 1] + hd[:, D + 1:D + 2]
    w = jnp.exp(_lrelu(a, 0.2))
    rid = (pl.program_id(0) * RBE
           + lax.broadcasted_iota(jnp.int32, (RBE, 1), 0))
    rloc = jnp.where(rid >= EPC, rid - EPC, rid)
    w = jnp.where(rloc < EC, w, 0.0)
    col = lax.broadcasted_iota(jnp.int32, (RBE, DP), 1)
    o = jnp.where(col < D, w * hs, 0.0)
    o_ref[...] = jnp.where(col == D, w, o)


_escale = pl.pallas_call(
    _escale_body,
    grid=(NTOT // RBE,),
    in_specs=[
        pl.BlockSpec((RBE, DP), lambda i: (i, 0)),
        pl.BlockSpec((RBE, DP), lambda i: (i, 0)),
    ],
    out_specs=pl.BlockSpec((RBE, DP), lambda i: (i, 0)),
    out_shape=jax.ShapeDtypeStruct((NTOT, DP), _f32),
)


def _heads_body(agg_ref, dp_ref, h2_ref, ws_ref, b_ref,
                wy1_ref, by1_ref, wy0_ref, by0_ref, wb_ref, bb_ref,
                wp1_ref, bp1_ref, wp2_ref, bp2_ref,
                qt_ref, fb_ref, tp_ref):
    rd = 1.0 / (dp_ref[...] + ws_ref[...] + 1e-16)
    z = rd * (agg_ref[...] + ws_ref[...] * h2_ref[...][:, :D]) + b_ref[...]
    p1 = jnp.dot(z, wy1_ref[...], preferred_element_type=_f32) + by1_ref[0, 0]
    p0 = jnp.dot(z, wy0_ref[...], preferred_element_type=_f32) + by0_ref[0, 0]
    q1 = _lrelu(p1, 0.01)
    q0 = _lrelu(p0, 0.01)
    col = lax.broadcasted_iota(jnp.int32, (RB, DP), 1)
    qt_ref[...] = jnp.where(col == 0, q1, jnp.where(col == 1, q0, 0.0))
    fb_ref[...] = jnp.dot(z, wb_ref[...], preferred_element_type=_f32) + bb_ref[...]
    t = _lrelu(jnp.dot(z, wp1_ref[...], preferred_element_type=_f32) + bp1_ref[...], 0.01)
    tp_ref[...] = _lrelu(
        jnp.dot(t, wp2_ref[...], preferred_element_type=_f32) + bp2_ref[...], 0.01)


_heads = pl.pallas_call(
    _heads_body,
    grid=(M // RB,),
    in_specs=[
        pl.BlockSpec((RB, D), lambda i: (i, 0)),
        pl.BlockSpec((RB, 1), lambda i: (i, 0)),
        pl.BlockSpec((RB, DP), lambda i: (i, 0)),
        pl.BlockSpec((RB, 1), lambda i: (i, 0)),
        pl.BlockSpec((1, D), lambda i: (0, 0)),
        pl.BlockSpec((D, 1), lambda i: (0, 0)),
        pl.BlockSpec((1, 1), lambda i: (0, 0)),
        pl.BlockSpec((D, 1), lambda i: (0, 0)),
        pl.BlockSpec((1, 1), lambda i: (0, 0)),
        pl.BlockSpec((D, 2), lambda i: (0, 0)),
        pl.BlockSpec((1, 2), lambda i: (0, 0)),
        pl.BlockSpec((D, D), lambda i: (0, 0)),
        pl.BlockSpec((1, D), lambda i: (0, 0)),
        pl.BlockSpec((D, 2), lambda i: (0, 0)),
        pl.BlockSpec((1, 2), lambda i: (0, 0)),
    ],
    out_specs=[
        pl.BlockSpec((RB, DP), lambda i: (i, 0)),
        pl.BlockSpec((RB, 2), lambda i: (i, 0)),
        pl.BlockSpec((RB, 2), lambda i: (i, 0)),
    ],
    out_shape=[
        jax.ShapeDtypeStruct((M, DP), _f32),
        jax.ShapeDtypeStruct((M, 2), _f32),
        jax.ShapeDtypeStruct((M, 2), _f32),
    ],
)


_sc_mesh = plsc.VectorSubcoreMesh(core_axis_name="c", subcore_axis_name="s")
_sc_params = pltpu.CompilerParams(needs_layout_passes=False)


def _make_rowgath(ntot):
    nper = ntot // NW
    nchunk = nper // CH

    @functools.partial(
        pl.kernel,
        out_type=jax.ShapeDtypeStruct((ntot, DP), _f32),
        mesh=_sc_mesh,
        scratch_types=[
            pltpu.VMEM((CH,), jnp.int32),
            pltpu.VMEM((CH, DP), _f32),
            pltpu.SemaphoreType.DMA,
        ],
        compiler_params=_sc_params,
    )
    def _rowgath(tab_hbm, idx_hbm, out_hbm, idx_v, rows_v, sem):
        cid = lax.axis_index("c")
        sid = lax.axis_index("s")
        base = (cid * NS + sid) * nper

        def body(j, carry):
            off = base + j * CH
            pltpu.sync_copy(idx_hbm.at[pl.ds(off, CH)], idx_v)
            pltpu.async_copy(tab_hbm.at[idx_v], rows_v, sem).wait()
            pltpu.sync_copy(rows_v, out_hbm.at[pl.ds(off, CH)])
            return carry

        lax.fori_loop(0, nchunk, body, 0)

    return _rowgath


_rowgath_e = _make_rowgath(NTOT)
_rowgath_q = _make_rowgath(G4)


@functools.partial(
    pl.kernel,
    out_type=jax.ShapeDtypeStruct((NC, N_PAD, DP), _f32),
    mesh=_sc_mesh,
    scratch_types=[
        pltpu.VMEM((CH,), jnp.int32),
        pltpu.VMEM((CH, DP), _f32),
        pltpu.VMEM_SHARED((N_PAD, DP), _f32),
        pltpu.SemaphoreType.DMA,
    ],
    compiler_params=_sc_params,
)
def _rowscat(rows_hbm, dst_hbm, agg_hbm, idx_v, rows_v, acc_sh, sem):
    cid = lax.axis_index("c")
    sid = lax.axis_index("s")
    z16 = jnp.zeros((16,), _f32)

    def zrow(r, c2):
        for c in range(DP // 16):
            rows_v[r, pl.ds(c * 16, 16)] = z16
        return c2

    lax.fori_loop(0, CH, zrow, 0)
    for t in range(ROWS_PS // CH):
        pltpu.sync_copy(rows_v, acc_sh.at[pl.ds(sid * ROWS_PS + t * CH, CH)])
    plsc.subcore_barrier()
    base = cid * EPC + sid * EPW

    def body(j, carry):
        off = base + j * CH
        pltpu.sync_copy(dst_hbm.at[pl.ds(off, CH)], idx_v)
        pltpu.sync_copy(rows_hbm.at[pl.ds(off, CH)], rows_v)
        pltpu.sync_copy(rows_v, acc_sh.at[idx_v], add=True)
        return carry

    lax.fori_loop(0, NCHUNK, body, 0)
    plsc.subcore_barrier()
    pltpu.sync_copy(acc_sh.at[pl.ds(sid * ROWS_PS, ROWS_PS)],
                    agg_hbm.at[cid, pl.ds(sid * ROWS_PS, ROWS_PS)])


def _pad_cols(w, cols=DP):
    return jnp.pad(w, ((0, 0), (0, cols - w.shape[1])))


def kernel(x, edge_index, fake_x, fake_edge_index, treat_idx, control_idx,
           W1, a_src1, a_dst1, b1, W2, a_src2, a_dst2, b2,
           Wy1, by1, Wy0, by0, Wb, bb, Wp1, bp1, Wp2, bp2):
    X = jnp.concatenate([x, fake_x], axis=0)
    npad = EPC - EC
    pad_s = (jnp.arange(npad, dtype=jnp.int32) * 7) % N
    pad_d = (jnp.arange(npad, dtype=jnp.int32) * 13) % N
    src3 = jnp.concatenate([
        edge_index[0], pad_s,
        fake_edge_index[0] + N, pad_s + N,
    ])
    dst3 = jnp.concatenate([
        edge_index[1], pad_d,
        fake_edge_index[1], pad_d,
    ])
    dst3g = jnp.concatenate([
        edge_index[1], pad_d,
        fake_edge_index[1] + N, pad_d + N,
    ])

    W1p = _pad_cols(W1)
    W2p = _pad_cols(W2)
    as1 = _pad_cols(a_src1.reshape(1, D))
    ad1 = _pad_cols(a_dst1.reshape(1, D))
    as2 = _pad_cols(a_src2.reshape(1, D))
    ad2 = _pad_cols(a_dst2.reshape(1, D))

    t1, ws1 = _dense1(X, W1p, as1, ad1)
    hs1 = _rowgath_e(t1, src3)
    hd1 = _rowgath_e(t1, dst3g)
    wr1 = _escale(hs1, hd1)
    agg1 = _rowscat(wr1, dst3)
    t2, ws2 = _dense2(agg1[:, :N, :D].reshape(M, D),
                      agg1[:, :N, D].reshape(M, 1),
                      t1, ws1, b1.reshape(1, D), W2p, as2, ad2)
    hs2 = _rowgath_e(t2, src3)
    hd2 = _rowgath_e(t2, dst3g)
    wr2 = _escale(hs2, hd2)
    agg2 = _rowscat(wr2, dst3)
    qt, fb, tp = _heads(agg2[:, :N, :D].reshape(M, D),
                        agg2[:, :N, D].reshape(M, 1),
                        t2, ws2, b2.reshape(1, D),
                        Wy1, by1.reshape(1, 1), Wy0, by0.reshape(1, 1),
                        Wb, bb.reshape(1, 2), Wp1, bp1.reshape(1, D),
                        Wp2, bp2.reshape(1, 2))

    pad_i = jnp.zeros((G_PAD - N_IDX,), jnp.int32)
    tpad = jnp.concatenate([treat_idx, pad_i])
    cpad = jnp.concatenate([control_idx, pad_i])
    g4 = jnp.concatenate([tpad, tpad + N, cpad, cpad + N])
    qr = _rowgath_q(qt, g4)
    y1 = qr[:N_IDX, 0]
    yc0 = qr[G_PAD:G_PAD + N_IDX, 1]
    y0 = qr[2 * G_PAD:2 * G_PAD + N_IDX, 1]
    yc1 = qr[3 * G_PAD:3 * G_PAD + N_IDX, 0]

    return (y1, yc0, y0, yc1, fb[:N], fb[N:], tp[:N])


# speedup vs baseline: 14.7715x; 1.1989x over previous
import functools

import jax
import jax.numpy as jnp
from jax import lax
from jax.experimental import pallas as pl
from jax.experimental.pallas import tpu as pltpu
from jax.experimental.pallas import tpu_sc as plsc

N = 10000
M = 2 * N
EC = 320000
D = 64
DP = 128
IN_DIM = 128

NC, NS = 2, 16
NW = NC * NS
CH = 128
NCHUNK = 160
EPW = NCHUNK * CH
EPC = NS * EPW
NTOT = NC * EPC
N_PAD = 10240
ROWS_PS = N_PAD // NS
N_IDX = 5000
G_PAD = 5120
G4 = 4 * G_PAD

_f32 = jnp.float32


RB = 2000
RBE = 4096


def _lrelu(x, slope):
    return jnp.where(x > 0, x, slope * x)


def _pack_tab(h, s, d, rb):
    col = lax.broadcasted_iota(jnp.int32, (rb, DP), 1)
    return jnp.where(col == D, s, jnp.where(col == D + 1, d, h))


def _dense1_body(x_ref, w_ref, as_ref, ad_ref, t_ref, ws_ref):
    h = jnp.dot(x_ref[...], w_ref[...], preferred_element_type=_f32)
    s = jnp.sum(h * as_ref[...], axis=1, keepdims=True)
    d = jnp.sum(h * ad_ref[...], axis=1, keepdims=True)
    t_ref[...] = _pack_tab(h, s, d, RB)
    ws_ref[...] = jnp.exp(_lrelu(s + d, 0.2))


_dense1 = pl.pallas_call(
    _dense1_body,
    grid=(M // RB,),
    in_specs=[
        pl.BlockSpec((RB, IN_DIM), lambda i: (i, 0)),
        pl.BlockSpec((IN_DIM, DP), lambda i: (0, 0)),
        pl.BlockSpec((1, DP), lambda i: (0, 0)),
        pl.BlockSpec((1, DP), lambda i: (0, 0)),
    ],
    out_specs=[
        pl.BlockSpec((RB, DP), lambda i: (i, 0)),
        pl.BlockSpec((RB, 1), lambda i: (i, 0)),
    ],
    out_shape=[
        jax.ShapeDtypeStruct((M, DP), _f32),
        jax.ShapeDtypeStruct((M, 1), _f32),
    ],
)


def _dense2_body(agg_ref, dp_ref, h1_ref, ws_ref, b_ref, w_ref,
                 as_ref, ad_ref, t_ref, ws2_ref):
    rd = 1.0 / (dp_ref[...] + ws_ref[...] + 1e-16)
    z = rd * (agg_ref[...] + ws_ref[...] * h1_ref[...][:, :D]) + b_ref[...]
    z = jnp.maximum(z, 0.0)
    h = jnp.dot(z, w_ref[...], preferred_element_type=_f32)
    s = jnp.sum(h * as_ref[...], axis=1, keepdims=True)
    d = jnp.sum(h * ad_ref[...], axis=1, keepdims=True)
    t_ref[...] = _pack_tab(h, s, d, RB)
    ws2_ref[...] = jnp.exp(_lrelu(s + d, 0.2))


_dense2 = pl.pallas_call(
    _dense2_body,
    grid=(M // RB,),
    in_specs=[
        pl.BlockSpec((RB, D), lambda i: (i, 0)),
        pl.BlockSpec((RB, 1), lambda i: (i, 0)),
        pl.BlockSpec((RB, DP), lambda i: (i, 0)),
        pl.BlockSpec((RB, 1), lambda i: (i, 0)),
        pl.BlockSpec((1, D), lambda i: (0, 0)),
        pl.BlockSpec((D, DP), lambda i: (0, 0)),
        pl.BlockSpec((1, DP), lambda i: (0, 0)),
        pl.BlockSpec((1, DP), lambda i: (0, 0)),
    ],
    out_specs=[
        pl.BlockSpec((RB, DP), lambda i: (i, 0)),
        pl.BlockSpec((RB, 1), lambda i: (i, 0)),
    ],
    out_shape=[
        jax.ShapeDtypeStruct((M, DP), _f32),
        jax.ShapeDtypeStruct((M, 1), _f32),
    ],
)


def _escale_body(hs_ref, hd_ref, o_ref):
    hs = hs_ref[...]
    hd = hd_ref[...]
    a = hs[:, D:D + 1] + hd[:, D + 1:D + 2]
    w = jnp.exp(_lrelu(a, 0.2))
    rid = (pl.program_id(0) * RBE
           + lax.broadcasted_iota(jnp.int32, (RBE, 1), 0))
    rloc = jnp.where(rid >= EPC, rid - EPC, rid)
    w = jnp.where(rloc < EC, w, 0.0)
    col = lax.broadcasted_iota(jnp.int32, (RBE, DP), 1)
    o = jnp.where(col < D, w * hs, 0.0)
    o_ref[...] = jnp.where(col == D, w, o)


_escale = pl.pallas_call(
    _escale_body,
    grid=(NTOT // RBE,),
    in_specs=[
        pl.BlockSpec((RBE, DP), lambda i: (i, 0)),
        pl.BlockSpec((RBE, DP), lambda i: (i, 0)),
    ],
    out_specs=pl.BlockSpec((RBE, DP), lambda i: (i, 0)),
    out_shape=jax.ShapeDtypeStruct((NTOT, DP), _f32),
)


def _heads_body(agg_ref, dp_ref, h2_ref, ws_ref, b_ref,
                wy1_ref, by1_ref, wy0_ref, by0_ref, wb_ref, bb_ref,
                wp1_ref, bp1_ref, wp2_ref, bp2_ref,
                qt_ref, fb_ref, tp_ref):
    rd = 1.0 / (dp_ref[...] + ws_ref[...] + 1e-16)
    z = rd * (agg_ref[...] + ws_ref[...] * h2_ref[...][:, :D]) + b_ref[...]
    p1 = jnp.dot(z, wy1_ref[...], preferred_element_type=_f32) + by1_ref[0, 0]
    p0 = jnp.dot(z, wy0_ref[...], preferred_element_type=_f32) + by0_ref[0, 0]
    q1 = _lrelu(p1, 0.01)
    q0 = _lrelu(p0, 0.01)
    col = lax.broadcasted_iota(jnp.int32, (RB, DP), 1)
    qt_ref[...] = jnp.where(col == 0, q1, jnp.where(col == 1, q0, 0.0))
    fb_ref[...] = jnp.dot(z, wb_ref[...], preferred_element_type=_f32) + bb_ref[...]
    t = _lrelu(jnp.dot(z, wp1_ref[...], preferred_element_type=_f32) + bp1_ref[...], 0.01)
    tp_ref[...] = _lrelu(
        jnp.dot(t, wp2_ref[...], preferred_element_type=_f32) + bp2_ref[...], 0.01)


_heads = pl.pallas_call(
    _heads_body,
    grid=(M // RB,),
    in_specs=[
        pl.BlockSpec((RB, D), lambda i: (i, 0)),
        pl.BlockSpec((RB, 1), lambda i: (i, 0)),
        pl.BlockSpec((RB, DP), lambda i: (i, 0)),
        pl.BlockSpec((RB, 1), lambda i: (i, 0)),
        pl.BlockSpec((1, D), lambda i: (0, 0)),
        pl.BlockSpec((D, 1), lambda i: (0, 0)),
        pl.BlockSpec((1, 1), lambda i: (0, 0)),
        pl.BlockSpec((D, 1), lambda i: (0, 0)),
        pl.BlockSpec((1, 1), lambda i: (0, 0)),
        pl.BlockSpec((D, 2), lambda i: (0, 0)),
        pl.BlockSpec((1, 2), lambda i: (0, 0)),
        pl.BlockSpec((D, D), lambda i: (0, 0)),
        pl.BlockSpec((1, D), lambda i: (0, 0)),
        pl.BlockSpec((D, 2), lambda i: (0, 0)),
        pl.BlockSpec((1, 2), lambda i: (0, 0)),
    ],
    out_specs=[
        pl.BlockSpec((RB, DP), lambda i: (i, 0)),
        pl.BlockSpec((RB, 2), lambda i: (i, 0)),
        pl.BlockSpec((RB, 2), lambda i: (i, 0)),
    ],
    out_shape=[
        jax.ShapeDtypeStruct((M, DP), _f32),
        jax.ShapeDtypeStruct((M, 2), _f32),
        jax.ShapeDtypeStruct((M, 2), _f32),
    ],
)


_sc_mesh = plsc.VectorSubcoreMesh(core_axis_name="c", subcore_axis_name="s")
_sc_params = pltpu.CompilerParams(needs_layout_passes=False)


def _make_rowgath(ntot):
    nper = ntot // NW
    nchunk = nper // CH

    @functools.partial(
        pl.kernel,
        out_type=jax.ShapeDtypeStruct((ntot, DP), _f32),
        mesh=_sc_mesh,
        scratch_types=[
            pltpu.VMEM((CH,), jnp.int32),
            pltpu.VMEM((CH,), jnp.int32),
            pltpu.VMEM((CH, DP), _f32),
            pltpu.VMEM((CH, DP), _f32),
            pltpu.SemaphoreType.DMA,
            pltpu.SemaphoreType.DMA,
        ],
        compiler_params=_sc_params,
    )
    def _rowgath(tab_hbm, idx_hbm, out_hbm,
                 idx0_v, idx1_v, rows0_v, rows1_v, sem0, sem1):
        cid = lax.axis_index("c")
        sid = lax.axis_index("s")
        base = (cid * NS + sid) * nper

        def body(j2, carry):
            off0 = base + (2 * j2) * CH
            off1 = off0 + CH
            pltpu.sync_copy(idx_hbm.at[pl.ds(off0, CH)], idx0_v)
            cp0 = pltpu.async_copy(tab_hbm.at[idx0_v], rows0_v, sem0)
            pltpu.sync_copy(idx_hbm.at[pl.ds(off1, CH)], idx1_v)
            cp1 = pltpu.async_copy(tab_hbm.at[idx1_v], rows1_v, sem1)
            cp0.wait()
            pltpu.sync_copy(rows0_v, out_hbm.at[pl.ds(off0, CH)])
            cp1.wait()
            pltpu.sync_copy(rows1_v, out_hbm.at[pl.ds(off1, CH)])
            return carry

        lax.fori_loop(0, nchunk // 2, body, 0)
        if nchunk % 2:
            off = base + (nchunk - 1) * CH
            pltpu.sync_copy(idx_hbm.at[pl.ds(off, CH)], idx0_v)
            pltpu.async_copy(tab_hbm.at[idx0_v], rows0_v, sem0).wait()
            pltpu.sync_copy(rows0_v, out_hbm.at[pl.ds(off, CH)])

    return _rowgath


_rowgath_e = _make_rowgath(NTOT)
_rowgath_q = _make_rowgath(G4)


@functools.partial(
    pl.kernel,
    out_type=jax.ShapeDtypeStruct((NC, N_PAD, DP), _f32),
    mesh=_sc_mesh,
    scratch_types=[
        pltpu.VMEM((CH,), jnp.int32),
        pltpu.VMEM((CH, DP), _f32),
        pltpu.VMEM_SHARED((N_PAD, DP), _f32),
        pltpu.SemaphoreType.DMA,
    ],
    compiler_params=_sc_params,
)
def _rowscat(rows_hbm, dst_hbm, agg_hbm, idx_v, rows_v, acc_sh, sem):
    cid = lax.axis_index("c")
    sid = lax.axis_index("s")
    z16 = jnp.zeros((16,), _f32)

    def zrow(r, c2):
        for c in range(DP // 16):
            rows_v[r, pl.ds(c * 16, 16)] = z16
        return c2

    lax.fori_loop(0, CH, zrow, 0)
    for t in range(ROWS_PS // CH):
        pltpu.sync_copy(rows_v, acc_sh.at[pl.ds(sid * ROWS_PS + t * CH, CH)])
    plsc.subcore_barrier()
    base = cid * EPC + sid * EPW

    def body(j, carry):
        off = base + j * CH
        pltpu.sync_copy(dst_hbm.at[pl.ds(off, CH)], idx_v)
        pltpu.sync_copy(rows_hbm.at[pl.ds(off, CH)], rows_v)
        pltpu.sync_copy(rows_v, acc_sh.at[idx_v], add=True)
        return carry

    lax.fori_loop(0, NCHUNK, body, 0)
    plsc.subcore_barrier()
    pltpu.sync_copy(acc_sh.at[pl.ds(sid * ROWS_PS, ROWS_PS)],
                    agg_hbm.at[cid, pl.ds(sid * ROWS_PS, ROWS_PS)])


def _pad_cols(w, cols=DP):
    return jnp.pad(w, ((0, 0), (0, cols - w.shape[1])))


def kernel(x, edge_index, fake_x, fake_edge_index, treat_idx, control_idx,
           W1, a_src1, a_dst1, b1, W2, a_src2, a_dst2, b2,
           Wy1, by1, Wy0, by0, Wb, bb, Wp1, bp1, Wp2, bp2):
    X = jnp.concatenate([x, fake_x], axis=0)
    npad = EPC - EC
    pad_s = (jnp.arange(npad, dtype=jnp.int32) * 7) % N
    pad_d = (jnp.arange(npad, dtype=jnp.int32) * 13) % N
    src3 = jnp.concatenate([
        edge_index[0], pad_s,
        fake_edge_index[0] + N, pad_s + N,
    ])
    dst3 = jnp.concatenate([
        edge_index[1], pad_d,
        fake_edge_index[1], pad_d,
    ])
    dst3g = jnp.concatenate([
        edge_index[1], pad_d,
        fake_edge_index[1] + N, pad_d + N,
    ])

    W1p = _pad_cols(W1)
    W2p = _pad_cols(W2)
    as1 = _pad_cols(a_src1.reshape(1, D))
    ad1 = _pad_cols(a_dst1.reshape(1, D))
    as2 = _pad_cols(a_src2.reshape(1, D))
    ad2 = _pad_cols(a_dst2.reshape(1, D))

    t1, ws1 = _dense1(X, W1p, as1, ad1)
    hs1 = _rowgath_e(t1, src3)
    hd1 = _rowgath_e(t1, dst3g)
    wr1 = _escale(hs1, hd1)
    agg1 = _rowscat(wr1, dst3)
    t2, ws2 = _dense2(agg1[:, :N, :D].reshape(M, D),
                      agg1[:, :N, D].reshape(M, 1),
                      t1, ws1, b1.reshape(1, D), W2p, as2, ad2)
    hs2 = _rowgath_e(t2, src3)
    hd2 = _rowgath_e(t2, dst3g)
    wr2 = _escale(hs2, hd2)
    agg2 = _rowscat(wr2, dst3)
    qt, fb, tp = _heads(agg2[:, :N, :D].reshape(M, D),
                        agg2[:, :N, D].reshape(M, 1),
                        t2, ws2, b2.reshape(1, D),
                        Wy1, by1.reshape(1, 1), Wy0, by0.reshape(1, 1),
                        Wb, bb.reshape(1, 2), Wp1, bp1.reshape(1, D),
                        Wp2, bp2.reshape(1, 2))

    pad_i = jnp.zeros((G_PAD - N_IDX,), jnp.int32)
    tpad = jnp.concatenate([treat_idx, pad_i])
    cpad = jnp.concatenate([control_idx, pad_i])
    g4 = jnp.concatenate([tpad, tpad + N, cpad, cpad + N])
    qr = _rowgath_q(qt, g4)
    y1 = qr[:N_IDX, 0]
    yc0 = qr[G_PAD:G_PAD + N_IDX, 1]
    y0 = qr[2 * G_PAD:2 * G_PAD + N_IDX, 1]
    yc1 = qr[3 * G_PAD:3 * G_PAD + N_IDX, 0]

    return (y1, yc0, y0, yc1, fb[:N], fb[N:], tp[:N])


# speedup vs baseline: 16.0091x; 1.0838x over previous
import functools

import jax
import jax.numpy as jnp
from jax import lax
from jax.experimental import pallas as pl
from jax.experimental.pallas import tpu as pltpu
from jax.experimental.pallas import tpu_sc as plsc

N = 10000
M = 2 * N
EC = 320000
D = 64
DP = 128
IN_DIM = 128

NC, NS = 2, 16
NW = NC * NS
CH = 128
NCHUNK = 160
EPW = NCHUNK * CH
EPC = NS * EPW
NTOT = NC * EPC
N_PAD = 10240
ROWS_PS = N_PAD // NS
N_IDX = 5000
G_PAD = 5120
G4 = 4 * G_PAD

_f32 = jnp.float32


RB = 2000
RBE = 4096


def _lrelu(x, slope):
    return jnp.where(x > 0, x, slope * x)


def _pack_tab(h, s, d, rb):
    col = lax.broadcasted_iota(jnp.int32, (rb, DP), 1)
    return jnp.where(col == D, s, jnp.where(col == D + 1, d, h))


def _dense1_body(x_ref, w_ref, as_ref, ad_ref, t_ref, ws_ref):
    h = jnp.dot(x_ref[...], w_ref[...], preferred_element_type=_f32)
    s = jnp.sum(h * as_ref[...], axis=1, keepdims=True)
    d = jnp.sum(h * ad_ref[...], axis=1, keepdims=True)
    t_ref[...] = _pack_tab(h, s, d, RB)
    ws_ref[...] = jnp.exp(_lrelu(s + d, 0.2))


_dense1 = pl.pallas_call(
    _dense1_body,
    grid=(M // RB,),
    in_specs=[
        pl.BlockSpec((RB, IN_DIM), lambda i: (i, 0)),
        pl.BlockSpec((IN_DIM, DP), lambda i: (0, 0)),
        pl.BlockSpec((1, DP), lambda i: (0, 0)),
        pl.BlockSpec((1, DP), lambda i: (0, 0)),
    ],
    out_specs=[
        pl.BlockSpec((RB, DP), lambda i: (i, 0)),
        pl.BlockSpec((RB, 1), lambda i: (i, 0)),
    ],
    out_shape=[
        jax.ShapeDtypeStruct((M, DP), _f32),
        jax.ShapeDtypeStruct((M, 1), _f32),
    ],
)


def _dense2_body(agg_ref, dp_ref, h1_ref, ws_ref, b_ref, w_ref,
                 as_ref, ad_ref, t_ref, ws2_ref):
    rd = 1.0 / (dp_ref[...] + ws_ref[...] + 1e-16)
    z = rd * (agg_ref[...] + ws_ref[...] * h1_ref[...][:, :D]) + b_ref[...]
    z = jnp.maximum(z, 0.0)
    h = jnp.dot(z, w_ref[...], preferred_element_type=_f32)
    s = jnp.sum(h * as_ref[...], axis=1, keepdims=True)
    d = jnp.sum(h * ad_ref[...], axis=1, keepdims=True)
    t_ref[...] = _pack_tab(h, s, d, RB)
    ws2_ref[...] = jnp.exp(_lrelu(s + d, 0.2))


_dense2 = pl.pallas_call(
    _dense2_body,
    grid=(M // RB,),
    in_specs=[
        pl.BlockSpec((RB, D), lambda i: (i, 0)),
        pl.BlockSpec((RB, 1), lambda i: (i, 0)),
        pl.BlockSpec((RB, DP), lambda i: (i, 0)),
        pl.BlockSpec((RB, 1), lambda i: (i, 0)),
        pl.BlockSpec((1, D), lambda i: (0, 0)),
        pl.BlockSpec((D, DP), lambda i: (0, 0)),
        pl.BlockSpec((1, DP), lambda i: (0, 0)),
        pl.BlockSpec((1, DP), lambda i: (0, 0)),
    ],
    out_specs=[
        pl.BlockSpec((RB, DP), lambda i: (i, 0)),
        pl.BlockSpec((RB, 1), lambda i: (i, 0)),
    ],
    out_shape=[
        jax.ShapeDtypeStruct((M, DP), _f32),
        jax.ShapeDtypeStruct((M, 1), _f32),
    ],
)


def _escale_body(hs_ref, hd_ref, o_ref):
    hs = hs_ref[...]
    hd = hd_ref[...]
    a = hs[:, D:D + 1] + hd[:, D + 1:D + 2]
    w = jnp.exp(_lrelu(a, 0.2))
    rid = (pl.program_id(0) * RBE
           + lax.broadcasted_iota(jnp.int32, (RBE, 1), 0))
    rloc = jnp.where(rid >= EPC, rid - EPC, rid)
    w = jnp.where(rloc < EC, w, 0.0)
    col = lax.broadcasted_iota(jnp.int32, (RBE, DP), 1)
    o = jnp.where(col < D, w * hs, 0.0)
    o_ref[...] = jnp.where(col == D, w, o)


_escale = pl.pallas_call(
    _escale_body,
    grid=(NTOT // RBE,),
    in_specs=[
        pl.BlockSpec((RBE, DP), lambda i: (i, 0)),
        pl.BlockSpec((RBE, DP), lambda i: (i, 0)),
    ],
    out_specs=pl.BlockSpec((RBE, DP), lambda i: (i, 0)),
    out_shape=jax.ShapeDtypeStruct((NTOT, DP), _f32),
)


def _heads_body(agg_ref, dp_ref, h2_ref, ws_ref, b_ref,
                wy1_ref, by1_ref, wy0_ref, by0_ref, wb_ref, bb_ref,
                wp1_ref, bp1_ref, wp2_ref, bp2_ref,
                qt_ref, fb_ref, tp_ref):
    rd = 1.0 / (dp_ref[...] + ws_ref[...] + 1e-16)
    z = rd * (agg_ref[...] + ws_ref[...] * h2_ref[...][:, :D]) + b_ref[...]
    p1 = jnp.dot(z, wy1_ref[...], preferred_element_type=_f32) + by1_ref[0, 0]
    p0 = jnp.dot(z, wy0_ref[...], preferred_element_type=_f32) + by0_ref[0, 0]
    q1 = _lrelu(p1, 0.01)
    q0 = _lrelu(p0, 0.01)
    col = lax.broadcasted_iota(jnp.int32, (RB, DP), 1)
    qt_ref[...] = jnp.where(col == 0, q1, jnp.where(col == 1, q0, 0.0))
    fb_ref[...] = jnp.dot(z, wb_ref[...], preferred_element_type=_f32) + bb_ref[...]
    t = _lrelu(jnp.dot(z, wp1_ref[...], preferred_element_type=_f32) + bp1_ref[...], 0.01)
    tp_ref[...] = _lrelu(
        jnp.dot(t, wp2_ref[...], preferred_element_type=_f32) + bp2_ref[...], 0.01)


_heads = pl.pallas_call(
    _heads_body,
    grid=(M // RB,),
    in_specs=[
        pl.BlockSpec((RB, D), lambda i: (i, 0)),
        pl.BlockSpec((RB, 1), lambda i: (i, 0)),
        pl.BlockSpec((RB, DP), lambda i: (i, 0)),
        pl.BlockSpec((RB, 1), lambda i: (i, 0)),
        pl.BlockSpec((1, D), lambda i: (0, 0)),
        pl.BlockSpec((D, 1), lambda i: (0, 0)),
        pl.BlockSpec((1, 1), lambda i: (0, 0)),
        pl.BlockSpec((D, 1), lambda i: (0, 0)),
        pl.BlockSpec((1, 1), lambda i: (0, 0)),
        pl.BlockSpec((D, 2), lambda i: (0, 0)),
        pl.BlockSpec((1, 2), lambda i: (0, 0)),
        pl.BlockSpec((D, D), lambda i: (0, 0)),
        pl.BlockSpec((1, D), lambda i: (0, 0)),
        pl.BlockSpec((D, 2), lambda i: (0, 0)),
        pl.BlockSpec((1, 2), lambda i: (0, 0)),
    ],
    out_specs=[
        pl.BlockSpec((RB, DP), lambda i: (i, 0)),
        pl.BlockSpec((RB, 2), lambda i: (i, 0)),
        pl.BlockSpec((RB, 2), lambda i: (i, 0)),
    ],
    out_shape=[
        jax.ShapeDtypeStruct((M, DP), _f32),
        jax.ShapeDtypeStruct((M, 2), _f32),
        jax.ShapeDtypeStruct((M, 2), _f32),
    ],
)


_sc_mesh = plsc.VectorSubcoreMesh(core_axis_name="c", subcore_axis_name="s")
_sc_params = pltpu.CompilerParams(needs_layout_passes=False)


def _make_rowgath(ntot):
    nper = ntot // NW
    nchunk = nper // CH

    @functools.partial(
        pl.kernel,
        out_type=jax.ShapeDtypeStruct((ntot, DP), _f32),
        mesh=_sc_mesh,
        scratch_types=[
            pltpu.VMEM((CH,), jnp.int32),
            pltpu.VMEM((CH,), jnp.int32),
            pltpu.VMEM((CH, DP), _f32),
            pltpu.VMEM((CH, DP), _f32),
            pltpu.SemaphoreType.DMA,
            pltpu.SemaphoreType.DMA,
        ],
        compiler_params=_sc_params,
    )
    def _rowgath(tab_hbm, idx_hbm, out_hbm,
                 idx0_v, idx1_v, rows0_v, rows1_v, sem0, sem1):
        cid = lax.axis_index("c")
        sid = lax.axis_index("s")
        base = (cid * NS + sid) * nper

        def body(j2, carry):
            off0 = base + (2 * j2) * CH
            off1 = off0 + CH
            pltpu.sync_copy(idx_hbm.at[pl.ds(off0, CH)], idx0_v)
            cp0 = pltpu.async_copy(tab_hbm.at[idx0_v], rows0_v, sem0)
            pltpu.sync_copy(idx_hbm.at[pl.ds(off1, CH)], idx1_v)
            cp1 = pltpu.async_copy(tab_hbm.at[idx1_v], rows1_v, sem1)
            cp0.wait()
            pltpu.sync_copy(rows0_v, out_hbm.at[pl.ds(off0, CH)])
            cp1.wait()
            pltpu.sync_copy(rows1_v, out_hbm.at[pl.ds(off1, CH)])
            return carry

        lax.fori_loop(0, nchunk // 2, body, 0)
        if nchunk % 2:
            off = base + (nchunk - 1) * CH
            pltpu.sync_copy(idx_hbm.at[pl.ds(off, CH)], idx0_v)
            pltpu.async_copy(tab_hbm.at[idx0_v], rows0_v, sem0).wait()
            pltpu.sync_copy(rows0_v, out_hbm.at[pl.ds(off, CH)])

    return _rowgath


_rowgath_e = _make_rowgath(NTOT)
_rowgath_q = _make_rowgath(G4)


@functools.partial(
    pl.kernel,
    out_type=jax.ShapeDtypeStruct((NC, N_PAD, DP), _f32),
    mesh=_sc_mesh,
    scratch_types=[
        pltpu.VMEM((CH,), jnp.int32),
        pltpu.VMEM((CH,), jnp.int32),
        pltpu.VMEM((CH, DP), _f32),
        pltpu.VMEM((CH, DP), _f32),
        pltpu.VMEM_SHARED((N_PAD, DP), _f32),
        pltpu.SemaphoreType.DMA,
        pltpu.SemaphoreType.DMA,
    ],
    compiler_params=_sc_params,
)
def _rowscat(rows_hbm, dst_hbm, agg_hbm,
             idx0_v, idx1_v, rows0_v, rows1_v, acc_sh, sem0, sem1):
    cid = lax.axis_index("c")
    sid = lax.axis_index("s")
    z16 = jnp.zeros((16,), _f32)

    def zrow(r, c2):
        for c in range(DP // 16):
            rows0_v[r, pl.ds(c * 16, 16)] = z16
        return c2

    lax.fori_loop(0, CH, zrow, 0)
    for t in range(ROWS_PS // CH):
        pltpu.sync_copy(rows0_v, acc_sh.at[pl.ds(sid * ROWS_PS + t * CH, CH)])
    plsc.subcore_barrier()
    base = cid * EPC + sid * EPW

    def body(j2, carry):
        off0 = base + (2 * j2) * CH
        off1 = off0 + CH
        pltpu.sync_copy(dst_hbm.at[pl.ds(off0, CH)], idx0_v)
        cp0 = pltpu.async_copy(rows_hbm.at[pl.ds(off0, CH)], rows0_v, sem0)
        pltpu.sync_copy(dst_hbm.at[pl.ds(off1, CH)], idx1_v)
        cp1 = pltpu.async_copy(rows_hbm.at[pl.ds(off1, CH)], rows1_v, sem1)
        cp0.wait()
        pltpu.sync_copy(rows0_v, acc_sh.at[idx0_v], add=True)
        cp1.wait()
        pltpu.sync_copy(rows1_v, acc_sh.at[idx1_v], add=True)
        return carry

    lax.fori_loop(0, NCHUNK // 2, body, 0)
    plsc.subcore_barrier()
    pltpu.sync_copy(acc_sh.at[pl.ds(sid * ROWS_PS, ROWS_PS)],
                    agg_hbm.at[cid, pl.ds(sid * ROWS_PS, ROWS_PS)])


def _pad_cols(w, cols=DP):
    return jnp.pad(w, ((0, 0), (0, cols - w.shape[1])))


def kernel(x, edge_index, fake_x, fake_edge_index, treat_idx, control_idx,
           W1, a_src1, a_dst1, b1, W2, a_src2, a_dst2, b2,
           Wy1, by1, Wy0, by0, Wb, bb, Wp1, bp1, Wp2, bp2):
    X = jnp.concatenate([x, fake_x], axis=0)
    npad = EPC - EC
    pad_s = (jnp.arange(npad, dtype=jnp.int32) * 7) % N
    pad_d = (jnp.arange(npad, dtype=jnp.int32) * 13) % N
    src3 = jnp.concatenate([
        edge_index[0], pad_s,
        fake_edge_index[0] + N, pad_s + N,
    ])
    dst3 = jnp.concatenate([
        edge_index[1], pad_d,
        fake_edge_index[1], pad_d,
    ])
    dst3g = jnp.concatenate([
        edge_index[1], pad_d,
        fake_edge_index[1] + N, pad_d + N,
    ])

    W1p = _pad_cols(W1)
    W2p = _pad_cols(W2)
    as1 = _pad_cols(a_src1.reshape(1, D))
    ad1 = _pad_cols(a_dst1.reshape(1, D))
    as2 = _pad_cols(a_src2.reshape(1, D))
    ad2 = _pad_cols(a_dst2.reshape(1, D))

    t1, ws1 = _dense1(X, W1p, as1, ad1)
    hs1 = _rowgath_e(t1, src3)
    hd1 = _rowgath_e(t1, dst3g)
    wr1 = _escale(hs1, hd1)
    agg1 = _rowscat(wr1, dst3)
    t2, ws2 = _dense2(agg1[:, :N, :D].reshape(M, D),
                      agg1[:, :N, D].reshape(M, 1),
                      t1, ws1, b1.reshape(1, D), W2p, as2, ad2)
    hs2 = _rowgath_e(t2, src3)
    hd2 = _rowgath_e(t2, dst3g)
    wr2 = _escale(hs2, hd2)
    agg2 = _rowscat(wr2, dst3)
    qt, fb, tp = _heads(agg2[:, :N, :D].reshape(M, D),
                        agg2[:, :N, D].reshape(M, 1),
                        t2, ws2, b2.reshape(1, D),
                        Wy1, by1.reshape(1, 1), Wy0, by0.reshape(1, 1),
                        Wb, bb.reshape(1, 2), Wp1, bp1.reshape(1, D),
                        Wp2, bp2.reshape(1, 2))

    pad_i = jnp.zeros((G_PAD - N_IDX,), jnp.int32)
    tpad = jnp.concatenate([treat_idx, pad_i])
    cpad = jnp.concatenate([control_idx, pad_i])
    g4 = jnp.concatenate([tpad, tpad + N, cpad, cpad + N])
    qr = _rowgath_q(qt, g4)
    y1 = qr[:N_IDX, 0]
    yc0 = qr[G_PAD:G_PAD + N_IDX, 1]
    y0 = qr[2 * G_PAD:2 * G_PAD + N_IDX, 1]
    yc1 = qr[3 * G_PAD:3 * G_PAD + N_IDX, 0]

    return (y1, yc0, y0, yc1, fb[:N], fb[N:], tp[:N])
